# Initial kernel scaffold; baseline (speedup 1.0000x reference)
import jax
import jax.numpy as jnp
from jax.experimental import pallas as pl


def kernel(x, edge_index, W, bias, prelu_w):
    raise NotImplementedError("write your pallas kernel here")


# speedup vs baseline: 20.5988x; 20.5988x over previous
import functools

import jax
import jax.numpy as jnp
from jax import lax
from jax.experimental import pallas as pl
from jax.experimental.pallas import tpu as pltpu
from jax.experimental.pallas import tpu_sc as plsc

N_NODES = 10000
CH = 128
N_EDGES = 320000

NC = 2
NS = 16
NW = NC * NS
LANES = 16

EDGES_PER_TILE = N_EDGES // NW
CHUNK = 80
NCHUNKS = EDGES_PER_TILE // CHUNK
STRIPE = 624
TAIL = N_NODES - NS * STRIPE


def _sc_mesh():
    return plsc.VectorSubcoreMesh(core_axis_name="c", subcore_axis_name="s")


def _hist_body(dst_hbm, hist_hbm, dst_v, hist_v):
    c = lax.axis_index("c")
    s = lax.axis_index("s")
    wid = c * NS + s

    def zero_step(i, _):
        hist_v[pl.ds(i * LANES, LANES)] = jnp.zeros((LANES,), jnp.float32)
        return 0

    lax.fori_loop(0, N_NODES // LANES, zero_step, 0)

    pltpu.sync_copy(dst_hbm.at[pl.ds(wid * EDGES_PER_TILE, EDGES_PER_TILE)], dst_v)

    ones = jnp.ones((LANES,), jnp.float32)

    def count_step(i, _):
        idx = dst_v[pl.ds(i * LANES, LANES)]
        plsc.addupdate_scatter(hist_v, [idx], ones)
        return 0

    lax.fori_loop(0, EDGES_PER_TILE // LANES, count_step, 0)

    pltpu.sync_copy(hist_v, hist_hbm.at[wid])


def _sc_histogram(dst):
    return pl.kernel(
        _hist_body,
        out_type=jax.ShapeDtypeStruct((NW, N_NODES), jnp.float32),
        mesh=_sc_mesh(),
        scratch_types=[
            pltpu.VMEM((N_EDGES // NW,), jnp.int32),
            pltpu.VMEM((N_NODES,), jnp.float32),
        ],
        compiler_params=pltpu.CompilerParams(needs_layout_passes=False),
    )(dst)


ROW_BLK = 1000


def _matmul_body(x_ref, w_ref, hist_ref, hp_ref):
    h = jnp.dot(x_ref[...], w_ref[...], preferred_element_type=jnp.float32)
    deg = 1.0 + jnp.sum(hist_ref[...], axis=1)
    dinv = lax.rsqrt(deg)
    hp_ref[...] = h * dinv[:, None]


def _tc_matmul_scale(x, W, hist):
    grid = (N_NODES // ROW_BLK,)
    return pl.pallas_call(
        _matmul_body,
        grid=grid,
        in_specs=[
            pl.BlockSpec((ROW_BLK, CH), lambda i: (i, 0)),
            pl.BlockSpec((CH, CH), lambda i: (0, 0)),
            pl.BlockSpec((ROW_BLK, NW), lambda i: (i, 0)),
        ],
        out_specs=pl.BlockSpec((ROW_BLK, CH), lambda i: (i, 0)),
        out_shape=jax.ShapeDtypeStruct((N_NODES, CH), jnp.float32),
    )(x, W, hist)


def _edge_body(hp_hbm, src_hbm, dst_hbm, part_hbm,
               src_v, dst_v, rows_v, acc_sh, sem):
    c = lax.axis_index("c")
    s = lax.axis_index("s")
    wid = c * NS + s

    def zero_step(i, _):
        for j in range(CH // LANES):
            rows_v[i, pl.ds(j * LANES, LANES)] = jnp.zeros((LANES,), jnp.float32)
        return 0

    lax.fori_loop(0, CHUNK, zero_step, 0)
    for r in range(STRIPE // CHUNK):
        pltpu.sync_copy(rows_v, acc_sh.at[pl.ds(s * STRIPE + r * CHUNK, CHUNK)])
    pltpu.sync_copy(rows_v.at[pl.ds(0, STRIPE % CHUNK)],
                    acc_sh.at[pl.ds(s * STRIPE + (STRIPE // CHUNK) * CHUNK,
                                    STRIPE % CHUNK)])

    @pl.when(s == NS - 1)
    def _():
        pltpu.sync_copy(rows_v.at[pl.ds(0, TAIL)],
                        acc_sh.at[pl.ds(NS * STRIPE, TAIL)])

    plsc.subcore_barrier()

    base = wid * EDGES_PER_TILE

    def chunk_step(i, _):
        off = base + i * CHUNK
        pltpu.sync_copy(src_hbm.at[pl.ds(off, CHUNK)], src_v)
        pltpu.sync_copy(dst_hbm.at[pl.ds(off, CHUNK)], dst_v)
        pltpu.async_copy(hp_hbm.at[src_v], rows_v, sem).wait()
        pltpu.sync_copy(rows_v, acc_sh.at[dst_v], add=True)
        return 0

    lax.fori_loop(0, NCHUNKS, chunk_step, 0)
    plsc.subcore_barrier()

    pltpu.sync_copy(acc_sh.at[pl.ds(s * STRIPE, STRIPE)],
                    part_hbm.at[c, pl.ds(s * STRIPE, STRIPE)])

    @pl.when(s == NS - 1)
    def _():
        pltpu.sync_copy(acc_sh.at[pl.ds(NS * STRIPE, TAIL)],
                        part_hbm.at[c, pl.ds(NS * STRIPE, TAIL)])


def _sc_edge_pass(hp, src, dst):
    return pl.kernel(
        _edge_body,
        out_type=jax.ShapeDtypeStruct((NC, N_NODES, CH), jnp.float32),
        mesh=_sc_mesh(),
        scratch_types=[
            pltpu.VMEM((CHUNK,), jnp.int32),
            pltpu.VMEM((CHUNK,), jnp.int32),
            pltpu.VMEM((CHUNK, CH), jnp.float32),
            pltpu.VMEM_SHARED((N_NODES, CH), jnp.float32),
            pltpu.SemaphoreType.DMA,
        ],
        compiler_params=pltpu.CompilerParams(needs_layout_passes=False),
    )(hp, src, dst)


def _final_body(part_ref, hp_ref, hist_ref, bias_ref, pw_ref, out_ref):
    deg = 1.0 + jnp.sum(hist_ref[...], axis=1)
    dinv = lax.rsqrt(deg)
    acc = part_ref[0] + part_ref[1] + hp_ref[...]
    o = acc * dinv[:, None] + bias_ref[...]
    out_ref[...] = jnp.where(o >= 0.0, o, pw_ref[...] * o)


def _tc_finalize(part, hp, hist, bias, prelu_w):
    grid = (N_NODES // ROW_BLK,)
    return pl.pallas_call(
        _final_body,
        grid=grid,
        in_specs=[
            pl.BlockSpec((NC, ROW_BLK, CH), lambda i: (0, i, 0)),
            pl.BlockSpec((ROW_BLK, CH), lambda i: (i, 0)),
            pl.BlockSpec((ROW_BLK, NW), lambda i: (i, 0)),
            pl.BlockSpec((1, CH), lambda i: (0, 0)),
            pl.BlockSpec((1, CH), lambda i: (0, 0)),
        ],
        out_specs=pl.BlockSpec((ROW_BLK, CH), lambda i: (i, 0)),
        out_shape=jax.ShapeDtypeStruct((N_NODES, CH), jnp.float32),
    )(part, hp, hist, bias, prelu_w)


def kernel(x, edge_index, W, bias, prelu_w):
    src = edge_index[0].astype(jnp.int32)
    dst = edge_index[1].astype(jnp.int32)
    hist = _sc_histogram(dst).T
    hp = _tc_matmul_scale(x, W, hist)
    part = _sc_edge_pass(hp, src, dst)
    return _tc_finalize(part, hp, hist,
                        bias.reshape(1, CH), prelu_w.reshape(1, CH))


# speedup vs baseline: 33.5956x; 1.6309x over previous
import functools

import jax
import jax.numpy as jnp
from jax import lax
from jax.experimental import pallas as pl
from jax.experimental.pallas import tpu as pltpu
from jax.experimental.pallas import tpu_sc as plsc

N_NODES = 10000
CH = 128
N_EDGES = 320000

NC = 2
NS = 16
NW = NC * NS
LANES = 16

EDGES_PER_TILE = N_EDGES // NW
CHUNK = 80
NCHUNKS = EDGES_PER_TILE // CHUNK
STRIPE = 624
TAIL = N_NODES - NS * STRIPE


def _sc_mesh():
    return plsc.VectorSubcoreMesh(core_axis_name="c", subcore_axis_name="s")


def _hist_body(dst_hbm, hist_hbm, dst_v, hist_v):
    c = lax.axis_index("c")
    s = lax.axis_index("s")
    wid = c * NS + s

    def zero_step(i, _):
        hist_v[pl.ds(i * LANES, LANES)] = jnp.zeros((LANES,), jnp.float32)
        return 0

    lax.fori_loop(0, N_NODES // LANES, zero_step, 0)

    pltpu.sync_copy(dst_hbm.at[pl.ds(wid * EDGES_PER_TILE, EDGES_PER_TILE)], dst_v)

    ones = jnp.ones((LANES,), jnp.float32)

    def count_step(i, _):
        idx = dst_v[pl.ds(i * LANES, LANES)]
        plsc.addupdate_scatter(hist_v, [idx], ones)
        return 0

    lax.fori_loop(0, EDGES_PER_TILE // LANES, count_step, 0)

    pltpu.sync_copy(hist_v, hist_hbm.at[wid])


def _sc_histogram(dst):
    return pl.kernel(
        _hist_body,
        out_type=jax.ShapeDtypeStruct((NW, N_NODES), jnp.float32),
        mesh=_sc_mesh(),
        scratch_types=[
            pltpu.VMEM((N_EDGES // NW,), jnp.int32),
            pltpu.VMEM((N_NODES,), jnp.float32),
        ],
        compiler_params=pltpu.CompilerParams(needs_layout_passes=False),
    )(dst)


ROW_BLK = 1000


def _matmul_body(x_ref, w_ref, hist_ref, hp_ref):
    h = jnp.dot(x_ref[...], w_ref[...], preferred_element_type=jnp.float32)
    deg = 1.0 + jnp.sum(hist_ref[...], axis=1)
    dinv = lax.rsqrt(deg)
    hp_ref[...] = h * dinv[:, None]


def _tc_matmul_scale(x, W, hist):
    grid = (N_NODES // ROW_BLK,)
    return pl.pallas_call(
        _matmul_body,
        grid=grid,
        in_specs=[
            pl.BlockSpec((ROW_BLK, CH), lambda i: (i, 0)),
            pl.BlockSpec((CH, CH), lambda i: (0, 0)),
            pl.BlockSpec((ROW_BLK, NW), lambda i: (i, 0)),
        ],
        out_specs=pl.BlockSpec((ROW_BLK, CH), lambda i: (i, 0)),
        out_shape=jax.ShapeDtypeStruct((N_NODES, CH), jnp.float32),
    )(x, W, hist)


def _edge_body(hp_hbm, eidx_hbm, part_hbm,
               islot, buf_a, buf_b, acc_sh,
               gs_a, gs_b, ss_a, ss_b, is_a, is_b):
    c = lax.axis_index("c")
    s = lax.axis_index("s")
    wid = c * NS + s

    def zero_step(i, _):
        for j in range(CH // LANES):
            buf_a[i, pl.ds(j * LANES, LANES)] = jnp.zeros((LANES,), jnp.float32)
        return 0

    lax.fori_loop(0, CHUNK, zero_step, 0)
    for r in range(STRIPE // CHUNK):
        pltpu.sync_copy(buf_a, acc_sh.at[pl.ds(s * STRIPE + r * CHUNK, CHUNK)])
    pltpu.sync_copy(buf_a.at[pl.ds(0, STRIPE % CHUNK)],
                    acc_sh.at[pl.ds(s * STRIPE + (STRIPE // CHUNK) * CHUNK,
                                    STRIPE % CHUNK)])

    @pl.when(s == NS - 1)
    def _():
        pltpu.sync_copy(buf_a.at[pl.ds(0, TAIL)],
                        acc_sh.at[pl.ds(NS * STRIPE, TAIL)])

    plsc.subcore_barrier()

    def idx_fire(chunk, sem):
        return pltpu.async_copy(eidx_hbm.at[wid, chunk], islot.at[chunk % 4], sem)

    def idx_wait(chunk, sem):
        pltpu.make_async_copy(eidx_hbm.at[wid, chunk],
                              islot.at[chunk % 4], sem).wait()

    def gather_start(chunk, buf, sem):
        return pltpu.async_copy(hp_hbm.at[islot.at[chunk % 4, 0]], buf, sem)

    def gather_wait(chunk, buf, sem):
        pltpu.make_async_copy(hp_hbm.at[islot.at[chunk % 4, 0]], buf, sem).wait()

    pltpu.sync_copy(eidx_hbm.at[wid, 0], islot.at[0])
    pltpu.sync_copy(eidx_hbm.at[wid, 1], islot.at[1])
    gather_start(0, buf_a, gs_a)
    gather_start(1, buf_b, gs_b)

    def pipe_step(i, _):
        ca = 2 * i
        cb = 2 * i + 1
        idx_fire(ca + 2, is_a)

        @pl.when(cb + 2 < NCHUNKS)
        def _():
            idx_fire(cb + 2, is_b)

        gather_wait(ca, buf_a, gs_a)
        da = pltpu.async_copy(buf_a, acc_sh.at[islot.at[ca % 4, 1]], ss_a,
                              add=True)
        gather_wait(cb, buf_b, gs_b)
        db = pltpu.async_copy(buf_b, acc_sh.at[islot.at[cb % 4, 1]], ss_b,
                              add=True)
        da.wait()
        idx_wait(ca + 2, is_a)
        gather_start(ca + 2, buf_a, gs_a)
        db.wait()

        @pl.when(cb + 2 < NCHUNKS)
        def _():
            idx_wait(cb + 2, is_b)
            gather_start(cb + 2, buf_b, gs_b)

        return 0

    lax.fori_loop(0, (NCHUNKS - 1) // 2, pipe_step, 0)
    gather_wait(NCHUNKS - 1, buf_a, gs_a)
    pltpu.sync_copy(buf_a, acc_sh.at[islot.at[(NCHUNKS - 1) % 4, 1]], add=True)
    plsc.subcore_barrier()

    pltpu.sync_copy(acc_sh.at[pl.ds(s * STRIPE, STRIPE)],
                    part_hbm.at[c, pl.ds(s * STRIPE, STRIPE)])

    @pl.when(s == NS - 1)
    def _():
        pltpu.sync_copy(acc_sh.at[pl.ds(NS * STRIPE, TAIL)],
                        part_hbm.at[c, pl.ds(NS * STRIPE, TAIL)])


def _sc_edge_pass(hp, src, dst):
    return pl.kernel(
        _edge_body,
        out_type=jax.ShapeDtypeStruct((NC, N_NODES, CH), jnp.float32),
        mesh=_sc_mesh(),
        scratch_types=[
            pltpu.VMEM((4, 2, CHUNK), jnp.int32),
            pltpu.VMEM((CHUNK, CH), jnp.float32),
            pltpu.VMEM((CHUNK, CH), jnp.float32),
            pltpu.VMEM_SHARED((N_NODES, CH), jnp.float32),
            pltpu.SemaphoreType.DMA,
            pltpu.SemaphoreType.DMA,
            pltpu.SemaphoreType.DMA,
            pltpu.SemaphoreType.DMA,
            pltpu.SemaphoreType.DMA,
            pltpu.SemaphoreType.DMA,
        ],
        compiler_params=pltpu.CompilerParams(needs_layout_passes=False),
    )(hp, jnp.stack([src.reshape(NW, NCHUNKS, CHUNK),
                     dst.reshape(NW, NCHUNKS, CHUNK)], axis=2))


def _final_body(part_ref, hp_ref, hist_ref, bias_ref, pw_ref, out_ref):
    deg = 1.0 + jnp.sum(hist_ref[...], axis=1)
    dinv = lax.rsqrt(deg)
    acc = part_ref[0] + part_ref[1] + hp_ref[...]
    o = acc * dinv[:, None] + bias_ref[...]
    out_ref[...] = jnp.where(o >= 0.0, o, pw_ref[...] * o)


def _tc_finalize(part, hp, hist, bias, prelu_w):
    grid = (N_NODES // ROW_BLK,)
    return pl.pallas_call(
        _final_body,
        grid=grid,
        in_specs=[
            pl.BlockSpec((NC, ROW_BLK, CH), lambda i: (0, i, 0)),
            pl.BlockSpec((ROW_BLK, CH), lambda i: (i, 0)),
            pl.BlockSpec((ROW_BLK, NW), lambda i: (i, 0)),
            pl.BlockSpec((1, CH), lambda i: (0, 0)),
            pl.BlockSpec((1, CH), lambda i: (0, 0)),
        ],
        out_specs=pl.BlockSpec((ROW_BLK, CH), lambda i: (i, 0)),
        out_shape=jax.ShapeDtypeStruct((N_NODES, CH), jnp.float32),
    )(part, hp, hist, bias, prelu_w)


def kernel(x, edge_index, W, bias, prelu_w):
    src = edge_index[0].astype(jnp.int32)
    dst = edge_index[1].astype(jnp.int32)
    hist = _sc_histogram(dst).T
    hp = _tc_matmul_scale(x, W, hist)
    part = _sc_edge_pass(hp, src, dst)
    return _tc_finalize(part, hp, hist,
                        bias.reshape(1, CH), prelu_w.reshape(1, CH))
